# baseline (speedup 1.0000x reference)
import functools

import jax
import jax.numpy as jnp
from jax import lax
from jax.experimental import pallas as pl
from jax.experimental.pallas import tpu as pltpu
from jax.experimental.pallas import tpu_sc as plsc

E = 8
K = 2
D = 2048
H = 2048
N = 2048
S = N * K
TM = 256
NT = S // TM
T = NT + E - 1

NC = 2
NS = 16
NW = NC * NS
LANES = 16
SCW = 128


def _gate_kernel(x_ref, wg_ref, bg_ref, out_ref, s0_ref, s1_ref, counts_ref):
    x = x_ref[...]
    logits = jax.lax.dot_general(
        x, wg_ref[...], (((1,), (1,)), ((), ())),
        preferred_element_type=jnp.float32)
    logits = logits + bg_ref[...]
    m = jnp.max(logits, axis=-1, keepdims=True)
    ex = jnp.exp(logits - m)
    sm = ex / jnp.sum(ex, axis=-1, keepdims=True)
    iota = jax.lax.broadcasted_iota(jnp.int32, (N, E), 1).astype(jnp.float32)
    big = jnp.float32(E)
    m1 = jnp.max(sm, axis=-1, keepdims=True)
    i1 = jnp.min(jnp.where(sm == m1, iota, big), axis=-1, keepdims=True)
    sm2 = jnp.where(iota == i1, -jnp.float32(1.0), sm)
    m2 = jnp.max(sm2, axis=-1, keepdims=True)
    i2 = jnp.min(jnp.where(sm2 == m2, iota, big), axis=-1, keepdims=True)

    onehot1 = (iota == i1).astype(jnp.float32)
    onehot2 = (iota == i2).astype(jnp.float32)
    B = onehot1 + onehot2
    r = jax.lax.broadcasted_iota(jnp.int32, (N, N), 0)
    c = jax.lax.broadcasted_iota(jnp.int32, (N, N), 1)
    L = (c < r).astype(jnp.bfloat16)
    C = jax.lax.dot_general(
        L, B.astype(jnp.bfloat16), (((1,), (0,)), ((), ())),
        preferred_element_type=jnp.float32)
    counts = jnp.sum(B, axis=0, keepdims=True)
    ei = jax.lax.broadcasted_iota(jnp.int32, (E, E), 0)
    ej = jax.lax.broadcasted_iota(jnp.int32, (E, E), 1)
    off = jnp.sum(jnp.where(ei < ej, counts.reshape(E, 1), 0.0),
                  axis=0, keepdims=True)
    rank1 = jnp.sum(C * onehot1, axis=1, keepdims=True)
    rank2 = jnp.sum((C + onehot1) * onehot2, axis=1, keepdims=True)
    pos1 = jnp.sum(off * onehot1, axis=1, keepdims=True) + rank1
    pos2 = jnp.sum(off * onehot2, axis=1, keepdims=True) + rank2

    out_ref[...] = pos1 * (iota == 0) + pos2 * (iota == 1)
    ones_w = jnp.ones((N, SCW), jnp.float32)
    s0_ref[...] = m1 * ones_w
    s1_ref[...] = m2 * ones_w
    counts_ref[...] = counts


def _gate(x, Wg, bg):
    out, s0, s1, counts = pl.pallas_call(
        _gate_kernel,
        out_shape=(jax.ShapeDtypeStruct((N, E), jnp.float32),
                   jax.ShapeDtypeStruct((N, SCW), jnp.float32),
                   jax.ShapeDtypeStruct((N, SCW), jnp.float32),
                   jax.ShapeDtypeStruct((1, E), jnp.float32)),
    )(x, Wg, bg.reshape(1, E))
    pos0 = out[:, 0].astype(jnp.int32)
    pos1 = out[:, 1].astype(jnp.int32)
    return pos0, pos1, s0, s1, counts.reshape(E).astype(jnp.int32)


_TOK_CHUNK = 16


def _dispatch_sc(x, pos0, pos1, s0, s1):
    mesh = plsc.VectorSubcoreMesh(core_axis_name="c", subcore_axis_name="s")

    tok_per_w = N // NW
    n_chunks = tok_per_w // _TOK_CHUNK

    @functools.partial(
        pl.kernel, mesh=mesh,
        out_type=(jax.ShapeDtypeStruct((S, D), jnp.float32),
                  jax.ShapeDtypeStruct((S, SCW), jnp.float32)),
        scratch_types=[
            pltpu.VMEM((2, _TOK_CHUNK, D), jnp.float32),
            pltpu.VMEM((2, _TOK_CHUNK, SCW), jnp.float32),
            pltpu.VMEM((2, _TOK_CHUNK, SCW), jnp.float32),
            pltpu.VMEM((2, _TOK_CHUNK), jnp.int32),
            pltpu.VMEM((2, _TOK_CHUNK), jnp.int32),
            pltpu.SemaphoreType.DMA,
            pltpu.SemaphoreType.DMA,
        ],
    )
    def k(x_hbm, p0_hbm, p1_hbm, s0_hbm, s1_hbm, xs_hbm, sc_hbm,
          rows_v, s0_v, s1_v, p0_v, p1_v, ssem0, ssem1):
        wid = lax.axis_index("s") * NC + lax.axis_index("c")
        ssems = (ssem0, ssem1)

        def load(j, b):
            tok0 = wid * tok_per_w + j * _TOK_CHUNK
            pltpu.sync_copy(x_hbm.at[pl.ds(tok0, _TOK_CHUNK)], rows_v.at[b])
            pltpu.sync_copy(s0_hbm.at[pl.ds(tok0, _TOK_CHUNK)], s0_v.at[b])
            pltpu.sync_copy(s1_hbm.at[pl.ds(tok0, _TOK_CHUNK)], s1_v.at[b])
            pltpu.sync_copy(p0_hbm.at[pl.ds(tok0, _TOK_CHUNK)], p0_v.at[b])
            pltpu.sync_copy(p1_hbm.at[pl.ds(tok0, _TOK_CHUNK)], p1_v.at[b])

        def start_scatters(b):
            return (pltpu.async_copy(rows_v.at[b], xs_hbm.at[p0_v.at[b]],
                                     ssems[b]),
                    pltpu.async_copy(rows_v.at[b], xs_hbm.at[p1_v.at[b]],
                                     ssems[b]),
                    pltpu.async_copy(s0_v.at[b], sc_hbm.at[p0_v.at[b]],
                                     ssems[b]),
                    pltpu.async_copy(s1_v.at[b], sc_hbm.at[p1_v.at[b]],
                                     ssems[b]))

        pending = {}
        load(0, 0)
        for j in range(n_chunks):
            b = j % 2
            nb = (j + 1) % 2
            if j >= 1:
                for h in pending[nb]:
                    h.wait()
            if j + 1 < n_chunks:
                load(j + 1, nb)
            pending[b] = start_scatters(b)
        for h in pending[(n_chunks - 1) % 2]:
            h.wait()

    return k(x, pos0, pos1, s0, s1)


def _gmm_kernel(info_ref, xs_ref, we_ref, be_ref, sc_ref, out_ref):
    t = pl.program_id(0)
    lo = info_ref[2, t]
    hi = info_ref[3, t]
    first = info_ref[4, t]
    rows = jax.lax.broadcasted_iota(jnp.int32, (TM, 1), 0)
    mask = (rows >= lo) & (rows < hi)
    x = xs_ref[...].astype(jnp.bfloat16)
    contrib = jax.lax.dot_general(
        x, we_ref[0].astype(jnp.bfloat16), (((1,), (1,)), ((), ())),
        preferred_element_type=jnp.float32)
    scale = jnp.where(mask, sc_ref[:, 0:1], jnp.float32(0.0))
    contrib = (contrib + be_ref[0]) * scale

    @pl.when(first == 1)
    def _():
        out_ref[...] = contrib

    @pl.when(first == 0)
    def _():
        out_ref[...] += contrib


def _grouped_matmul(info, x_sorted, We, be, scale_sorted):
    grid_spec = pltpu.PrefetchScalarGridSpec(
        num_scalar_prefetch=1,
        grid=(T,),
        in_specs=[
            pl.BlockSpec((TM, D), lambda t, info: (info[0, t], 0)),
            pl.BlockSpec((1, H, D), lambda t, info: (info[1, t], 0, 0)),
            pl.BlockSpec((1, 1, H), lambda t, info: (info[1, t], 0, 0)),
            pl.BlockSpec((TM, SCW), lambda t, info: (info[0, t], 0)),
        ],
        out_specs=pl.BlockSpec((TM, H), lambda t, info: (info[0, t], 0)),
    )
    return pl.pallas_call(
        _gmm_kernel,
        grid_spec=grid_spec,
        out_shape=jax.ShapeDtypeStruct((S, H), jnp.float32),
    )(info, x_sorted, We, be.reshape(E, 1, H), scale_sorted)


def _visit_maps(counts):
    offsets = jnp.concatenate(
        [jnp.zeros((1,), jnp.int32), jnp.cumsum(counts, dtype=jnp.int32)])
    first_tile = offsets[:E] // TM
    last_tile = jnp.maximum(offsets[1:] - 1, 0) // TM
    tiles_g = jnp.where(counts > 0, last_tile - first_tile + 1, 0)
    vb = jnp.concatenate(
        [jnp.zeros((1,), jnp.int32), jnp.cumsum(tiles_g, dtype=jnp.int32)])
    t_act = vb[E]
    tt = jnp.arange(T, dtype=jnp.int32)
    gid = jnp.sum((tt[:, None] >= vb[None, 1:]).astype(jnp.int32), axis=1)
    gid = jnp.clip(gid, 0, E - 1)
    valid = tt < t_act
    mt = first_tile[gid] + (tt - vb[gid])
    mt = jnp.where(valid, mt, NT - 1)
    glo = offsets[gid]
    ghi = offsets[gid + 1]
    lo = jnp.where(valid, jnp.clip(glo - mt * TM, 0, TM), 0)
    hi = jnp.where(valid, jnp.clip(ghi - mt * TM, 0, TM), 0)
    prev_mt = jnp.concatenate([jnp.full((1,), -1, jnp.int32), mt[:-1]])
    first = (valid & (mt != prev_mt)).astype(jnp.int32)
    return jnp.stack([mt, gid, lo, hi, first])


_CTOK = 8


def _combine_sc(y_sorted, pos0, pos1):
    mesh = plsc.VectorSubcoreMesh(core_axis_name="c", subcore_axis_name="s")
    tok_per_w = N // NW
    n_chunks = tok_per_w // _CTOK

    @functools.partial(
        pl.kernel, mesh=mesh,
        out_type=jax.ShapeDtypeStruct((N, H), jnp.float32),
        scratch_types=[
            pltpu.VMEM((2, K * _CTOK, H), jnp.float32),
            pltpu.VMEM((2, _CTOK, H), jnp.float32),
            pltpu.VMEM((2, K * _CTOK), jnp.int32),
            pltpu.SemaphoreType.DMA,
            pltpu.SemaphoreType.DMA,
            pltpu.SemaphoreType.DMA,
            pltpu.SemaphoreType.DMA,
        ],
    )
    def k(y_hbm, p0_hbm, p1_hbm, out_hbm, yrows_v, out_v, pc_v,
          gsem0, gsem1, wsem0, wsem1):
        wid = lax.axis_index("s") * NC + lax.axis_index("c")
        gsems = (gsem0, gsem1)
        wsems = (wsem0, wsem1)

        def start_gather(j, b):
            tok0 = wid * tok_per_w + j * _CTOK
            pltpu.sync_copy(p0_hbm.at[pl.ds(tok0, _CTOK)],
                            pc_v.at[b, pl.ds(0, _CTOK)])
            pltpu.sync_copy(p1_hbm.at[pl.ds(tok0, _CTOK)],
                            pc_v.at[b, pl.ds(_CTOK, _CTOK)])
            return pltpu.async_copy(y_hbm.at[pc_v.at[b]], yrows_v.at[b],
                                    gsems[b])

        def drain_write(j):
            b = j % 2
            tok0 = wid * tok_per_w + j * _CTOK
            pltpu.make_async_copy(
                out_v.at[b], out_hbm.at[pl.ds(tok0, _CTOK)],
                wsems[b]).wait()

        g = start_gather(0, 0)
        for j in range(n_chunks):
            b = j % 2
            nb = (j + 1) % 2
            g.wait()
            if j + 1 < n_chunks:
                g = start_gather(j + 1, nb)
            if j >= 2:
                drain_write(j - 2)
            for t in range(_CTOK):
                def body(cc, _, t=t, b=b):
                    for u in range(8):
                        col = (cc * 8 + u) * LANES
                        y0 = yrows_v[b, t, pl.ds(col, LANES)]
                        y1 = yrows_v[b, _CTOK + t, pl.ds(col, LANES)]
                        out_v[b, t, pl.ds(col, LANES)] = y0 + y1
                    return ()

                lax.fori_loop(0, H // LANES // 8, body, ())
            tok0 = wid * tok_per_w + j * _CTOK
            pltpu.async_copy(out_v.at[b],
                             out_hbm.at[pl.ds(tok0, _CTOK)], wsems[b])
        drain_write(n_chunks - 2)
        drain_write(n_chunks - 1)

    return k(y_sorted, pos0, pos1)


def kernel(x, Wg, bg, We, be):
    pos0, pos1, s0, s1, counts = _gate(x, Wg, bg)
    info = _visit_maps(counts)
    x_sorted, scale_sorted = _dispatch_sc(x, pos0, pos1, s0, s1)
    y_sorted = _grouped_matmul(info, x_sorted, We, be, scale_sorted)
    return _combine_sc(y_sorted, pos0, pos1)


# speedup vs baseline: 1.0040x; 1.0040x over previous
import functools

import jax
import jax.numpy as jnp
from jax import lax
from jax.experimental import pallas as pl
from jax.experimental.pallas import tpu as pltpu
from jax.experimental.pallas import tpu_sc as plsc

E = 8
K = 2
D = 2048
H = 2048
N = 2048
S = N * K
TM = 256
NT = S // TM
T = NT + E - 1

NC = 2
NS = 16
NW = NC * NS
LANES = 16
SCW = 128


def _gate_kernel(x_ref, wg_ref, bg_ref, out_ref, s0_ref, s1_ref, counts_ref):
    x = x_ref[...]
    logits = jax.lax.dot_general(
        x, wg_ref[...], (((1,), (1,)), ((), ())),
        preferred_element_type=jnp.float32)
    logits = logits + bg_ref[...]
    m = jnp.max(logits, axis=-1, keepdims=True)
    ex = jnp.exp(logits - m)
    sm = ex / jnp.sum(ex, axis=-1, keepdims=True)
    iota = jax.lax.broadcasted_iota(jnp.int32, (N, E), 1).astype(jnp.float32)
    big = jnp.float32(E)
    m1 = jnp.max(sm, axis=-1, keepdims=True)
    i1 = jnp.min(jnp.where(sm == m1, iota, big), axis=-1, keepdims=True)
    sm2 = jnp.where(iota == i1, -jnp.float32(1.0), sm)
    m2 = jnp.max(sm2, axis=-1, keepdims=True)
    i2 = jnp.min(jnp.where(sm2 == m2, iota, big), axis=-1, keepdims=True)

    onehot1 = (iota == i1).astype(jnp.float32)
    onehot2 = (iota == i2).astype(jnp.float32)
    B = onehot1 + onehot2
    r = jax.lax.broadcasted_iota(jnp.int32, (N, N), 0)
    c = jax.lax.broadcasted_iota(jnp.int32, (N, N), 1)
    L = (c < r).astype(jnp.bfloat16)
    C = jax.lax.dot_general(
        L, B.astype(jnp.bfloat16), (((1,), (0,)), ((), ())),
        preferred_element_type=jnp.float32)
    counts = jnp.sum(B, axis=0, keepdims=True)
    ei = jax.lax.broadcasted_iota(jnp.int32, (E, E), 0)
    ej = jax.lax.broadcasted_iota(jnp.int32, (E, E), 1)
    off = jnp.sum(jnp.where(ei < ej, counts.reshape(E, 1), 0.0),
                  axis=0, keepdims=True)
    rank1 = jnp.sum(C * onehot1, axis=1, keepdims=True)
    rank2 = jnp.sum((C + onehot1) * onehot2, axis=1, keepdims=True)
    pos1 = jnp.sum(off * onehot1, axis=1, keepdims=True) + rank1
    pos2 = jnp.sum(off * onehot2, axis=1, keepdims=True) + rank2

    out_ref[...] = pos1 * (iota == 0) + pos2 * (iota == 1)
    ones_w = jnp.ones((N, SCW), jnp.float32)
    s0_ref[...] = m1 * ones_w
    s1_ref[...] = m2 * ones_w
    counts_ref[...] = counts


def _gate(x, Wg, bg):
    out, s0, s1, counts = pl.pallas_call(
        _gate_kernel,
        out_shape=(jax.ShapeDtypeStruct((N, E), jnp.float32),
                   jax.ShapeDtypeStruct((N, SCW), jnp.float32),
                   jax.ShapeDtypeStruct((N, SCW), jnp.float32),
                   jax.ShapeDtypeStruct((1, E), jnp.float32)),
    )(x, Wg, bg.reshape(1, E))
    pos0 = out[:, 0].astype(jnp.int32)
    pos1 = out[:, 1].astype(jnp.int32)
    return pos0, pos1, s0, s1, counts.reshape(E).astype(jnp.int32)


_TOK_CHUNK = 16


def _dispatch_sc(x, pos0, pos1, s0, s1):
    mesh = plsc.VectorSubcoreMesh(core_axis_name="c", subcore_axis_name="s")

    tok_per_w = N // NW
    n_chunks = tok_per_w // _TOK_CHUNK

    @functools.partial(
        pl.kernel, mesh=mesh,
        out_type=(jax.ShapeDtypeStruct((S, D), jnp.float32),
                  jax.ShapeDtypeStruct((S, SCW), jnp.float32)),
        scratch_types=[
            pltpu.VMEM((2, _TOK_CHUNK, D), jnp.float32),
            pltpu.VMEM((2, _TOK_CHUNK, SCW), jnp.float32),
            pltpu.VMEM((2, _TOK_CHUNK, SCW), jnp.float32),
            pltpu.VMEM((2, _TOK_CHUNK), jnp.int32),
            pltpu.VMEM((2, _TOK_CHUNK), jnp.int32),
            pltpu.SemaphoreType.DMA,
            pltpu.SemaphoreType.DMA,
        ],
    )
    def k(x_hbm, p0_hbm, p1_hbm, s0_hbm, s1_hbm, xs_hbm, sc_hbm,
          rows_v, s0_v, s1_v, p0_v, p1_v, ssem0, ssem1):
        wid = lax.axis_index("s") * NC + lax.axis_index("c")
        ssems = (ssem0, ssem1)

        def load(j, b):
            tok0 = wid * tok_per_w + j * _TOK_CHUNK
            pltpu.sync_copy(x_hbm.at[pl.ds(tok0, _TOK_CHUNK)], rows_v.at[b])
            pltpu.sync_copy(s0_hbm.at[pl.ds(tok0, _TOK_CHUNK)], s0_v.at[b])
            pltpu.sync_copy(s1_hbm.at[pl.ds(tok0, _TOK_CHUNK)], s1_v.at[b])
            pltpu.sync_copy(p0_hbm.at[pl.ds(tok0, _TOK_CHUNK)], p0_v.at[b])
            pltpu.sync_copy(p1_hbm.at[pl.ds(tok0, _TOK_CHUNK)], p1_v.at[b])

        def start_scatters(b):
            return (pltpu.async_copy(rows_v.at[b], xs_hbm.at[p0_v.at[b]],
                                     ssems[b]),
                    pltpu.async_copy(rows_v.at[b], xs_hbm.at[p1_v.at[b]],
                                     ssems[b]),
                    pltpu.async_copy(s0_v.at[b], sc_hbm.at[p0_v.at[b]],
                                     ssems[b]),
                    pltpu.async_copy(s1_v.at[b], sc_hbm.at[p1_v.at[b]],
                                     ssems[b]))

        pending = {}
        load(0, 0)
        for j in range(n_chunks):
            b = j % 2
            nb = (j + 1) % 2
            if j >= 1:
                for h in pending[nb]:
                    h.wait()
            if j + 1 < n_chunks:
                load(j + 1, nb)
            pending[b] = start_scatters(b)
        for h in pending[(n_chunks - 1) % 2]:
            h.wait()

    return k(x, pos0, pos1, s0, s1)


def _gmm_kernel(info_ref, xs_ref, we_ref, be_ref, sc_ref, out_ref):
    t = pl.program_id(0)
    lo = info_ref[2, t]
    hi = info_ref[3, t]
    first = info_ref[4, t]
    @pl.when(hi > lo)
    def _():
        rows = jax.lax.broadcasted_iota(jnp.int32, (TM, 1), 0)
        mask = (rows >= lo) & (rows < hi)
        x = xs_ref[...].astype(jnp.bfloat16)
        contrib = jax.lax.dot_general(
            x, we_ref[0].astype(jnp.bfloat16), (((1,), (1,)), ((), ())),
            preferred_element_type=jnp.float32)
        scale = jnp.where(mask, sc_ref[:, 0:1], jnp.float32(0.0))
        contrib = (contrib + be_ref[0]) * scale
        out_ref[...] = jnp.where(first == 1, contrib,
                                 out_ref[...] + contrib)


def _grouped_matmul(info, x_sorted, We, be3, scale_sorted, hblk, hw):
    grid_spec = pltpu.PrefetchScalarGridSpec(
        num_scalar_prefetch=1,
        grid=(T,),
        in_specs=[
            pl.BlockSpec((TM, D), lambda t, info: (info[0, t], 0)),
            pl.BlockSpec((1, hw, D),
                         lambda t, info: (info[1, t], hblk, 0)),
            pl.BlockSpec((1, 1, hw),
                         lambda t, info: (info[1, t], 0, hblk)),
            pl.BlockSpec((TM, SCW), lambda t, info: (info[0, t], 0)),
        ],
        out_specs=pl.BlockSpec((TM, hw), lambda t, info: (info[0, t], 0)),
    )
    return pl.pallas_call(
        _gmm_kernel,
        grid_spec=grid_spec,
        out_shape=jax.ShapeDtypeStruct((S, hw), jnp.float32),
    )(info, x_sorted, We, be3, scale_sorted)


def _visit_maps(counts):
    offsets = jnp.concatenate(
        [jnp.zeros((1,), jnp.int32), jnp.cumsum(counts, dtype=jnp.int32)])
    first_tile = offsets[:E] // TM
    last_tile = jnp.maximum(offsets[1:] - 1, 0) // TM
    tiles_g = jnp.where(counts > 0, last_tile - first_tile + 1, 0)
    vb = jnp.concatenate(
        [jnp.zeros((1,), jnp.int32), jnp.cumsum(tiles_g, dtype=jnp.int32)])
    t_act = vb[E]
    tt = jnp.arange(T, dtype=jnp.int32)
    gid = jnp.sum((tt[:, None] >= vb[None, 1:]).astype(jnp.int32), axis=1)
    gid = jnp.clip(gid, 0, E - 1)
    valid = tt < t_act
    mt = first_tile[gid] + (tt - vb[gid])
    mt = jnp.where(valid, mt, NT - 1)
    glo = offsets[gid]
    ghi = offsets[gid + 1]
    lo = jnp.where(valid, jnp.clip(glo - mt * TM, 0, TM), 0)
    hi = jnp.where(valid, jnp.clip(ghi - mt * TM, 0, TM), 0)
    prev_mt = jnp.concatenate([jnp.full((1,), -1, jnp.int32), mt[:-1]])
    first = (valid & (mt != prev_mt)).astype(jnp.int32)
    return jnp.stack([mt, gid, lo, hi, first])


_CTOK = 8


def _combine_sc(y_sorted, pos0, pos1, hw):
    mesh = plsc.VectorSubcoreMesh(core_axis_name="c", subcore_axis_name="s")
    tok_per_w = N // NW
    n_chunks = tok_per_w // _CTOK

    @functools.partial(
        pl.kernel, mesh=mesh,
        out_type=jax.ShapeDtypeStruct((N, hw), jnp.float32),
        scratch_types=[
            pltpu.VMEM((2, K * _CTOK, hw), jnp.float32),
            pltpu.VMEM((2, _CTOK, hw), jnp.float32),
            pltpu.VMEM((2, K * _CTOK), jnp.int32),
            pltpu.SemaphoreType.DMA,
            pltpu.SemaphoreType.DMA,
            pltpu.SemaphoreType.DMA,
            pltpu.SemaphoreType.DMA,
        ],
    )
    def k(y_hbm, p0_hbm, p1_hbm, out_hbm, yrows_v, out_v, pc_v,
          gsem0, gsem1, wsem0, wsem1):
        wid = lax.axis_index("s") * NC + lax.axis_index("c")
        gsems = (gsem0, gsem1)
        wsems = (wsem0, wsem1)

        def start_gather(j, b):
            tok0 = wid * tok_per_w + j * _CTOK
            pltpu.sync_copy(p0_hbm.at[pl.ds(tok0, _CTOK)],
                            pc_v.at[b, pl.ds(0, _CTOK)])
            pltpu.sync_copy(p1_hbm.at[pl.ds(tok0, _CTOK)],
                            pc_v.at[b, pl.ds(_CTOK, _CTOK)])
            return pltpu.async_copy(y_hbm.at[pc_v.at[b]], yrows_v.at[b],
                                    gsems[b])

        def drain_write(j):
            b = j % 2
            tok0 = wid * tok_per_w + j * _CTOK
            pltpu.make_async_copy(
                out_v.at[b], out_hbm.at[pl.ds(tok0, _CTOK)],
                wsems[b]).wait()

        g = start_gather(0, 0)
        for j in range(n_chunks):
            b = j % 2
            nb = (j + 1) % 2
            g.wait()
            if j + 1 < n_chunks:
                g = start_gather(j + 1, nb)
            if j >= 2:
                drain_write(j - 2)
            for t in range(_CTOK):
                def body(cc, _, t=t, b=b):
                    for u in range(8):
                        col = (cc * 8 + u) * LANES
                        y0 = yrows_v[b, t, pl.ds(col, LANES)]
                        y1 = yrows_v[b, _CTOK + t, pl.ds(col, LANES)]
                        out_v[b, t, pl.ds(col, LANES)] = y0 + y1
                    return ()

                lax.fori_loop(0, hw // LANES // 8, body, ())
            tok0 = wid * tok_per_w + j * _CTOK
            pltpu.async_copy(out_v.at[b],
                             out_hbm.at[pl.ds(tok0, _CTOK)], wsems[b])
        drain_write(n_chunks - 2)
        drain_write(n_chunks - 1)

    return k(y_sorted, pos0, pos1)


def kernel(x, Wg, bg, We, be):
    pos0, pos1, s0, s1, counts = _gate(x, Wg, bg)
    info = _visit_maps(counts)
    x_sorted, scale_sorted = _dispatch_sc(x, pos0, pos1, s0, s1)
    be3 = be.reshape(E, 1, H)
    y_sorted = _grouped_matmul(info, x_sorted, We, be3, scale_sorted, 0, H)
    return _combine_sc(y_sorted, pos0, pos1, H)
